# baseline (speedup 1.0000x reference)
import functools

import jax
import jax.numpy as jnp
from jax import lax
from jax.experimental import pallas as pl
from jax.experimental.pallas import tpu as pltpu
from jax.experimental.pallas import tpu_sc as plsc

N_NODES = 10000
N_EDGES = 320000
D_FEAT = 128
D_EDGE = 16
IN_SIZE = D_FEAT + D_EDGE
HID = IN_SIZE * 2
CW = 128
CNT_COL = HID - 2 * CW

NSUB = 16
EB = 80
E_PAD = 327680
EBLKS = E_PAD // EB
BLKS_PER_CORE = EBLKS // 2
BLKS_PER_TILE = BLKS_PER_CORE // NSUB
PAIRS = BLKS_PER_TILE // 2
CHUNK_BLKS = 16
CHUNK_PAIRS = CHUNK_BLKS // 2
N_PAD = 10240
ROWS_PER_TILE = N_PAD // NSUB

HID_PAD = 3 * CW
PASSES = ((0, 8, 8), (CW, 8, 8), (2 * CW, 2, 3))

NODE_BLK = 1000
EDGE_BLK = 8192


def _xw_body(x_ref, w_ref, b_ref, o0_ref, o1_ref, o2_ref):
    r = jnp.dot(x_ref[...], w_ref[...], preferred_element_type=jnp.float32)
    r = r + b_ref[...]
    n = r.shape[0]
    o0_ref[...] = r[:, :CW]
    o1_ref[...] = r[:, CW:2 * CW]
    o2_ref[...] = jnp.concatenate(
        [r[:, 2 * CW:], jnp.ones((n, 1), jnp.float32),
         jnp.zeros((n, CW - CNT_COL - 1), jnp.float32)], axis=1)


def _ew_body(ea_ref, w_ref, o_ref):
    o_ref[...] = jnp.dot(ea_ref[...], w_ref[...],
                         preferred_element_type=jnp.float32)


_SC_MESH = plsc.VectorSubcoreMesh(core_axis_name="c", subcore_axis_name="s")


@functools.partial(
    pl.kernel,
    out_type=[
        jax.ShapeDtypeStruct((2, N_PAD, CW), jnp.float32),
        jax.ShapeDtypeStruct((2, N_PAD, CW), jnp.float32),
        jax.ShapeDtypeStruct((2, N_PAD, CW), jnp.float32),
    ],
    mesh=_SC_MESH,
    scratch_types=[
        pltpu.VMEM((CHUNK_BLKS, EB), jnp.int32),
        pltpu.VMEM((CHUNK_BLKS, EB), jnp.int32),
        pltpu.VMEM((EB, CW), jnp.float32),
        pltpu.VMEM((EB, CW), jnp.float32),
        pltpu.VMEM((EB, CW), jnp.float32),
        pltpu.VMEM((EB, CW), jnp.float32),
        pltpu.VMEM_SHARED((N_PAD, CW), jnp.float32),
        pltpu.SemaphoreType.DMA,
        pltpu.SemaphoreType.DMA,
        pltpu.SemaphoreType.DMA,
        pltpu.SemaphoreType.DMA,
        pltpu.SemaphoreType.DMA,
        pltpu.SemaphoreType.DMA,
    ],
)
def _sc_aggregate(xw0_h, xw1_h, xw2_h, ew_h, row_h, col_h, zrow_h,
                  p0_o, p1_o, p2_o,
                  row_v, col_v, gbufa, gbufb, ebufa, ebufb, acc,
                  sga, sgb, sea, seb, ssa, ssb):
    c = lax.axis_index("c")
    s = lax.axis_index("s")
    r0 = pl.multiple_of(s * ROWS_PER_TILE, 8)
    b0 = pl.multiple_of(c * BLKS_PER_CORE + s * BLKS_PER_TILE, 8)

    def run_pass(xw_h, ew_off, ew_nv, nv, out_ref):
        pltpu.sync_copy(zrow_h, acc.at[pl.ds(r0, ROWS_PER_TILE)])
        plsc.subcore_barrier()

        def inbound(blk, jrow, gbuf, ebuf, sg, se):
            e0 = pl.multiple_of((b0 + blk) * EB, 8)
            pltpu.async_copy(xw_h.at[row_v.at[jrow]], gbuf, sg)
            pltpu.async_copy(ew_h.at[pl.ds(e0, EB), pl.ds(ew_off, CW)],
                             ebuf, se)

        def wait_inbound(gbuf, ebuf, sg, se):
            pltpu.make_async_copy(xw_h.at[row_v.at[0]], gbuf, sg).wait()
            pltpu.make_async_copy(
                ew_h.at[pl.ds(0, EB), pl.ds(ew_off, CW)], ebuf, se).wait()

        def compute(gbuf, ebuf):
            def row_body(j, cr):
                for k in range(nv):
                    sl = pl.ds(k * 16, 16)
                    g = gbuf[j, sl]
                    if k < ew_nv:
                        g = g + ebuf[j, sl]
                    gbuf[j, sl] = jnp.maximum(g, 0.0)
                return cr

            lax.fori_loop(0, EB, row_body, 0)

        def wait_scatter(gbuf, ss):
            pltpu.make_async_copy(gbuf, acc.at[col_v.at[0]], ss).wait()

        def body(i, carry):
            refill = lax.rem(i, CHUNK_PAIRS) == 0
            jba = lax.rem(2 * i, CHUNK_BLKS)

            @pl.when(jnp.logical_and(refill, i > 0))
            def _():
                wait_scatter(gbufa, ssa)
                wait_scatter(gbufb, ssb)

            @pl.when(refill)
            def _():
                ck = pl.multiple_of(
                    b0 + lax.div(i, CHUNK_PAIRS) * CHUNK_BLKS, 8)
                pltpu.sync_copy(row_h.at[pl.ds(ck, CHUNK_BLKS)], row_v)
                pltpu.sync_copy(col_h.at[pl.ds(ck, CHUNK_BLKS)], col_v)
                inbound(2 * i, 0, gbufa, ebufa, sga, sea)
                inbound(2 * i + 1, 1, gbufb, ebufb, sgb, seb)

            wait_inbound(gbufa, ebufa, sga, sea)
            compute(gbufa, ebufa)
            pltpu.async_copy(gbufa, acc.at[col_v.at[jba]], ssa, add=True)
            wait_inbound(gbufb, ebufb, sgb, seb)
            compute(gbufb, ebufb)
            pltpu.async_copy(gbufb, acc.at[col_v.at[jba + 1]], ssb, add=True)

            @pl.when(jnp.logical_and(lax.rem(i, CHUNK_PAIRS)
                                     != CHUNK_PAIRS - 1, i < PAIRS - 1))
            def _():
                wait_scatter(gbufa, ssa)
                inbound(2 * i + 2, jba + 2, gbufa, ebufa, sga, sea)
                wait_scatter(gbufb, ssb)
                inbound(2 * i + 3, jba + 3, gbufb, ebufb, sgb, seb)

            return carry

        lax.fori_loop(0, PAIRS, body, 0)
        wait_scatter(gbufa, ssa)
        wait_scatter(gbufb, ssb)
        plsc.subcore_barrier()
        pltpu.sync_copy(acc.at[pl.ds(r0, ROWS_PER_TILE)],
                        out_ref.at[c, pl.ds(r0, ROWS_PER_TILE)])

    run_pass(xw0_h, PASSES[0][0], PASSES[0][1], PASSES[0][2], p0_o)
    run_pass(xw1_h, PASSES[1][0], PASSES[1][1], PASSES[1][2], p1_o)
    run_pass(xw2_h, PASSES[2][0], PASSES[2][1], PASSES[2][2], p2_o)


def _node_body(x_ref, p0a_ref, p0b_ref, p1a_ref, p1b_ref, p2a_ref, p2b_ref,
               w2a_ref, w2b_ref, w2c_ref, b2_ref,
               w3x_ref, w3m_ref, b3_ref, w4_ref, b4_ref, o_ref):
    r0 = p0a_ref[0] + p0b_ref[0]
    r1 = p1a_ref[0] + p1b_ref[0]
    r2 = p2a_ref[0] + p2b_ref[0]
    cnt = r2[:, CNT_COL:CNT_COL + 1]
    inv = 1.0 / jnp.maximum(cnt, 1.0)
    gate = jnp.minimum(cnt, 1.0)
    m = jnp.dot(r0 * inv, w2a_ref[...], preferred_element_type=jnp.float32)
    m = m + jnp.dot(r1 * inv, w2b_ref[...], preferred_element_type=jnp.float32)
    m = m + jnp.dot(r2[:, :CNT_COL] * inv, w2c_ref[...],
                    preferred_element_type=jnp.float32)
    m = m + gate * b2_ref[...]
    h = jnp.dot(x_ref[...], w3x_ref[...], preferred_element_type=jnp.float32)
    h = h + jnp.dot(m, w3m_ref[...], preferred_element_type=jnp.float32)
    h = jnp.maximum(h + b3_ref[...], 0.0)
    o_ref[...] = jnp.dot(h, w4_ref[...],
                         preferred_element_type=jnp.float32) + b4_ref[...]


@jax.jit
def kernel(x, edge_index, edge_attr, W1, b1, W2, b2, W3, b3, W4, b4):
    row = jnp.pad(edge_index[0].astype(jnp.int32),
                  (0, E_PAD - N_EDGES)).reshape(EBLKS, EB)
    col = jnp.pad(edge_index[1].astype(jnp.int32), (0, E_PAD - N_EDGES),
                  constant_values=N_NODES).reshape(EBLKS, EB)

    xw0, xw1, xw2 = pl.pallas_call(
        _xw_body,
        grid=(N_NODES // NODE_BLK,),
        in_specs=[
            pl.BlockSpec((NODE_BLK, D_FEAT), lambda i: (i, 0)),
            pl.BlockSpec((D_FEAT, HID), lambda i: (0, 0)),
            pl.BlockSpec((1, HID), lambda i: (0, 0)),
        ],
        out_specs=[
            pl.BlockSpec((NODE_BLK, CW), lambda i: (i, 0)),
            pl.BlockSpec((NODE_BLK, CW), lambda i: (i, 0)),
            pl.BlockSpec((NODE_BLK, CW), lambda i: (i, 0)),
        ],
        out_shape=[
            jax.ShapeDtypeStruct((N_NODES, CW), jnp.float32),
            jax.ShapeDtypeStruct((N_NODES, CW), jnp.float32),
            jax.ShapeDtypeStruct((N_NODES, CW), jnp.float32),
        ],
    )(x, W1[:D_FEAT], b1.reshape(1, HID))

    w1e_pad = jnp.concatenate(
        [W1[D_FEAT:], jnp.zeros((D_EDGE, HID_PAD - HID), jnp.float32)], axis=1)
    ew = pl.pallas_call(
        _ew_body,
        grid=(N_EDGES // EDGE_BLK,),
        in_specs=[
            pl.BlockSpec((EDGE_BLK, D_EDGE), lambda i: (i, 0)),
            pl.BlockSpec((D_EDGE, HID_PAD), lambda i: (0, 0)),
        ],
        out_specs=pl.BlockSpec((EDGE_BLK, HID_PAD), lambda i: (i, 0)),
        out_shape=jax.ShapeDtypeStruct((E_PAD, HID_PAD), jnp.float32),
    )(edge_attr, w1e_pad)

    zrow = jnp.zeros((ROWS_PER_TILE, CW), jnp.float32)
    p0, p1, p2 = _sc_aggregate(xw0, xw1, xw2, ew, row, col, zrow)

    def blk(i):
        return (i, 0)

    out = pl.pallas_call(
        _node_body,
        grid=(N_NODES // NODE_BLK,),
        in_specs=[
            pl.BlockSpec((NODE_BLK, D_FEAT), blk),
            pl.BlockSpec((1, NODE_BLK, CW), lambda i: (0, i, 0)),
            pl.BlockSpec((1, NODE_BLK, CW), lambda i: (1, i, 0)),
            pl.BlockSpec((1, NODE_BLK, CW), lambda i: (0, i, 0)),
            pl.BlockSpec((1, NODE_BLK, CW), lambda i: (1, i, 0)),
            pl.BlockSpec((1, NODE_BLK, CW), lambda i: (0, i, 0)),
            pl.BlockSpec((1, NODE_BLK, CW), lambda i: (1, i, 0)),
            pl.BlockSpec((CW, HID), lambda i: (0, 0)),
            pl.BlockSpec((CW, HID), lambda i: (0, 0)),
            pl.BlockSpec((CNT_COL, HID), lambda i: (0, 0)),
            pl.BlockSpec((1, HID), lambda i: (0, 0)),
            pl.BlockSpec((D_FEAT, IN_SIZE), lambda i: (0, 0)),
            pl.BlockSpec((HID, IN_SIZE), lambda i: (0, 0)),
            pl.BlockSpec((1, IN_SIZE), lambda i: (0, 0)),
            pl.BlockSpec((IN_SIZE, D_FEAT), lambda i: (0, 0)),
            pl.BlockSpec((1, D_FEAT), lambda i: (0, 0)),
        ],
        out_specs=pl.BlockSpec((NODE_BLK, D_FEAT), blk),
        out_shape=jax.ShapeDtypeStruct((N_NODES, D_FEAT), jnp.float32),
    )(x, p0, p0, p1, p1, p2, p2, W2[:CW], W2[CW:2 * CW], W2[2 * CW:],
      b2.reshape(1, HID), W3[:D_FEAT], W3[D_FEAT:], b3.reshape(1, IN_SIZE),
      W4, b4.reshape(1, D_FEAT))
    return out
